# baseline (speedup 1.0000x reference)
import jax
import jax.numpy as jnp
from jax import lax
from jax.experimental import pallas as pl
from jax.experimental.pallas import tpu as pltpu
from jax.experimental.pallas import tpu_sc as plsc

N = 10000
D = 128
E = 160000
NC = 2
NS = 16
CHUNK = 128
NCHUNK = 40
EPAD = NC * NS * NCHUNK * CHUNK
NPAD = 10240
RPW = NPAD // NS
BT = 1024
PK = 2
CH0 = 52
CH1 = 28
EPS = 1e-5


def _sc_mesh():
    return plsc.VectorSubcoreMesh(core_axis_name="c", subcore_axis_name="s")


def _counts6(srcs, dsts, zeros_h):
    def body(i0, i1, i2, i3, i4, i5, eye_h, zeros_h, cnt_out,
             idx_a, ones_v, rows, acc, ssem):
        c = lax.axis_index("c")
        s = lax.axis_index("s")
        idxs = (i0, i1, i2, i3, i4, i5)
        pltpu.sync_copy(zeros_h, rows)
        for t in range(RPW // CHUNK):
            pltpu.sync_copy(rows, acc.at[pl.ds(s * RPW + t * CHUNK, CHUNK)])
        plsc.subcore_barrier()
        for k in range(6):
            pltpu.sync_copy(eye_h.at[k], ones_v)
            pltpu.sync_copy(idxs[k].at[c, s], idx_a)

            def grp_body(g, carry, k=k):
                sds = [pltpu.async_copy(ones_v, acc.at[idx_a.at[g * PK + q]],
                                        ssem, add=True)
                       for q in range(PK)]
                for sd in sds:
                    sd.wait()
                return carry

            lax.fori_loop(0, NCHUNK // PK, grp_body, 0)
        plsc.subcore_barrier()
        for t in range(RPW // CHUNK):
            rowsl = pl.ds(s * RPW + t * CHUNK, CHUNK)
            pltpu.sync_copy(acc.at[rowsl], rows)
            pltpu.sync_copy(rows, cnt_out.at[c, rowsl])

    f = pl.kernel(
        body,
        out_type=jax.ShapeDtypeStruct((NC, NPAD, D), jnp.float32),
        mesh=_sc_mesh(),
        scratch_types=[
            pltpu.VMEM((NCHUNK, CHUNK), jnp.int32),
            pltpu.VMEM((CHUNK, D), jnp.float32),
            pltpu.VMEM((CHUNK, D), jnp.float32),
            pltpu.VMEM_SHARED((NPAD, D), jnp.float32),
            pltpu.SemaphoreType.DMA,
        ],
    )
    mask = jnp.repeat(jnp.eye(6, 8, dtype=jnp.float32), 16, axis=1)
    eye_h = jnp.broadcast_to(mask[:, None, :], (6, CHUNK, D))
    return f(srcs[0], dsts[0], srcs[1], dsts[1], srcs[2], dsts[2],
             eye_h, zeros_h)


def _scatter3(y0, y1, y2, srcs, dsts, zeros_h):
    def body(y0, y1, y2, s0, d0, s1, d1, s2, d2, zeros_h, o0, o1, o2,
             idx_sa, idx_da, r0, r1, acc,
             g0, g1, ssem):
        c = lax.axis_index("c")
        s = lax.axis_index("s")
        ys = (y0, y1, y2)
        sps = (s0, s1, s2)
        dps = (d0, d1, d2)
        outs = (o0, o1, o2)
        rbufs = (r0, r1)
        gsems = (g0, g1)
        for r in range(3):
            pltpu.sync_copy(zeros_h, r0)
            for t in range(RPW // CHUNK):
                pltpu.sync_copy(r0,
                                acc.at[pl.ds(s * RPW + t * CHUNK, CHUNK)])
            plsc.subcore_barrier()
            pltpu.sync_copy(sps[r].at[c, s], idx_sa)
            pltpu.sync_copy(dps[r].at[c, s], idx_da)

            def grp_body(g, carry, r=r):
                gds = [pltpu.async_copy(ys[r].at[idx_sa.at[g * PK + q]],
                                        rbufs[q], gsems[q])
                       for q in range(PK)]
                sds = []
                for q in range(PK):
                    gds[q].wait()
                    sds.append(pltpu.async_copy(
                        rbufs[q], acc.at[idx_da.at[g * PK + q]],
                        ssem, add=True))
                for sd in sds:
                    sd.wait()
                return carry

            ngrp = jnp.where(c == 0, CH0 // PK, CH1 // PK)
            lax.fori_loop(0, ngrp, grp_body, 0)
            plsc.subcore_barrier()
            for t in range(RPW // CHUNK):
                rowsl = pl.ds(s * RPW + t * CHUNK, CHUNK)
                pltpu.sync_copy(acc.at[rowsl], r0)
                pltpu.sync_copy(r0, outs[r].at[c, rowsl])

    f = pl.kernel(
        body,
        out_type=[jax.ShapeDtypeStruct((NC, NPAD, D), jnp.float32)] * 3,
        mesh=_sc_mesh(),
        scratch_types=[
            pltpu.VMEM((CH0, CHUNK), jnp.int32),
            pltpu.VMEM((CH0, CHUNK), jnp.int32),
            pltpu.VMEM((CHUNK, D), jnp.float32),
            pltpu.VMEM((CHUNK, D), jnp.float32),
            pltpu.VMEM_SHARED((NPAD, D), jnp.float32),
            pltpu.SemaphoreType.DMA,
            pltpu.SemaphoreType.DMA,
            pltpu.SemaphoreType.DMA,
        ],
    )
    return f(y0, y1, y2, srcs[0], dsts[0], srcs[1], dsts[1],
             srcs[2], dsts[2], zeros_h)


_BLK = pl.BlockSpec((BT, D), lambda i: (i, 0))
_SCL = pl.BlockSpec((6, BT, 16), lambda i: (0, i, 0))
_FULL = pl.BlockSpec(memory_space=pltpu.VMEM)


def _scales_body(cnt_ref, scl_ref):
    tot = cnt_ref[0] + cnt_ref[1]
    for k in range(6):
        scl_ref[k] = lax.rsqrt(jnp.maximum(tot[:, 16 * k:16 * (k + 1)], 1.0))


def _scales(cnt):
    return pl.pallas_call(
        _scales_body,
        grid=(NPAD // BT,),
        in_specs=[pl.BlockSpec((NC, BT, D), lambda i: (0, i, 0))],
        out_specs=_SCL,
        out_shape=jax.ShapeDtypeStruct((6, NPAD, 16), jnp.float32),
    )(cnt)


def _premm_body(h_ref, scl_ref, w0_ref, w1_ref, w2_ref,
                y0_ref, y1_ref, y2_ref):
    h = h_ref[...]
    for r, (w_ref, y_ref) in enumerate(((w0_ref, y0_ref),
                                        (w1_ref, y1_ref),
                                        (w2_ref, y2_ref))):
        y_ref[...] = jnp.dot(h * scl_ref[2 * r, :, 0:1], w_ref[...],
                             preferred_element_type=jnp.float32,
                             precision=lax.Precision.HIGHEST)


def _premm(h, scl, w0, w1, w2):
    shp = jax.ShapeDtypeStruct((NPAD, D), jnp.float32)
    return pl.pallas_call(
        _premm_body,
        grid=(NPAD // BT,),
        in_specs=[_BLK, _SCL, _FULL, _FULL, _FULL],
        out_specs=[_BLK, _BLK, _BLK],
        out_shape=[shp, shp, shp],
    )(h, scl, w0, w1, w2)


def _fc_body(a0_ref, a1_ref, a2_ref, scl_ref, b0_ref, b1_ref, b2_ref,
             wfc_ref, bfc_ref, u_ref, stats_ref, sacc_ref):
    i = pl.program_id(0)
    z = (b0_ref[...] + b1_ref[...] + b2_ref[...])[None, :]
    for r, a_ref in enumerate((a0_ref, a1_ref, a2_ref)):
        z = z + (a_ref[0] + a_ref[1]) * scl_ref[2 * r + 1, :, 0:1]
    u = jnp.dot(z, wfc_ref[...], preferred_element_type=jnp.float32,
                             precision=lax.Precision.HIGHEST)
    u = jnp.maximum(u + bfc_ref[...][None, :], 0.0)
    u_ref[...] = u
    ids = i * BT + lax.broadcasted_iota(jnp.int32, (BT, 1), 0)
    um = jnp.where(ids < N, u, 0.0)

    @pl.when(i == 0)
    def _():
        sacc_ref[...] = jnp.zeros_like(sacc_ref)

    sacc_ref[0:1, :] += jnp.sum(um, axis=0, keepdims=True)
    sacc_ref[1:2, :] += jnp.sum(um * um, axis=0, keepdims=True)

    @pl.when(i == pl.num_programs(0) - 1)
    def _():
        stats_ref[...] = sacc_ref[...]


def _fc_stage(a0, a1, a2, scl, b0, b1, b2, wfc, bfc):
    acc_spec = pl.BlockSpec((NC, BT, D), lambda i: (0, i, 0))
    return pl.pallas_call(
        _fc_body,
        grid=(NPAD // BT,),
        in_specs=[acc_spec, acc_spec, acc_spec, _SCL,
                  _FULL, _FULL, _FULL, _FULL, _FULL],
        out_specs=[_BLK, pl.BlockSpec((8, 128), lambda i: (0, 0))],
        out_shape=[jax.ShapeDtypeStruct((NPAD, D), jnp.float32),
                   jax.ShapeDtypeStruct((8, 128), jnp.float32)],
        scratch_shapes=[pltpu.VMEM((8, 128), jnp.float32)],
    )(a0, a1, a2, scl, b0, b1, b2, wfc, bfc)


def _bn(u, stats, gamma, beta):
    mean = stats[0:1, :] * (1.0 / N)
    var = stats[1:2, :] * (1.0 / N) - mean * mean
    return gamma[None, :] * (u - mean) * lax.rsqrt(var + EPS) + beta[None, :]


def _bn_premm_body(u_ref, stats_ref, g_ref, be_ref, scl_ref,
                   w0_ref, w1_ref, w2_ref, y0_ref, y1_ref, y2_ref):
    h = _bn(u_ref[...], stats_ref[...], g_ref[...], be_ref[...])
    for r, (w_ref, y_ref) in enumerate(((w0_ref, y0_ref),
                                        (w1_ref, y1_ref),
                                        (w2_ref, y2_ref))):
        y_ref[...] = jnp.dot(h * scl_ref[2 * r, :, 0:1], w_ref[...],
                             preferred_element_type=jnp.float32,
                             precision=lax.Precision.HIGHEST)


def _bn_premm(u, stats, gamma, beta, scl, w0, w1, w2):
    shp = jax.ShapeDtypeStruct((NPAD, D), jnp.float32)
    return pl.pallas_call(
        _bn_premm_body,
        grid=(NPAD // BT,),
        in_specs=[_BLK, _FULL, _FULL, _FULL, _SCL, _FULL, _FULL, _FULL],
        out_specs=[_BLK, _BLK, _BLK],
        out_shape=[shp, shp, shp],
    )(u, stats, gamma, beta, scl, w0, w1, w2)


def _bn_final_body(u_ref, stats_ref, g_ref, be_ref, out_ref):
    out_ref[...] = _bn(u_ref[...], stats_ref[...], g_ref[...], be_ref[...])


def _bn_final(u, stats, gamma, beta):
    blk = pl.BlockSpec((1000, D), lambda i: (i, 0))
    return pl.pallas_call(
        _bn_final_body,
        grid=(N // 1000,),
        in_specs=[blk, _FULL, _FULL, _FULL],
        out_specs=blk,
        out_shape=jax.ShapeDtypeStruct((N, D), jnp.float32),
    )(u, stats, gamma, beta)


def kernel(x, edge_index_seq, edge_index_knn, edge_index_dis,
           W0_seq, b0_seq, W0_knn, b0_knn, W0_dis, b0_dis, Wfc0, bfc0,
           gamma0, beta0,
           W1_seq, b1_seq, W1_knn, b1_knn, W1_dis, b1_dis, Wfc1, bfc1,
           gamma1, beta1):
    xp = jnp.pad(x, ((0, NPAD - N), (0, 0)))
    pad = jnp.full((EPAD - E,), N, jnp.int32)
    e0 = NS * CH0 * CHUNK

    def _asym(flat):
        a0 = flat[:e0].reshape(NS, CH0, CHUNK)
        a1 = flat[e0:].reshape(NS, CH1, CHUNK)
        a1 = jnp.concatenate(
            [a1, jnp.zeros((NS, CH0 - CH1, CHUNK), jnp.int32)], axis=1)
        return jnp.stack([a0, a1])

    srcs, dsts, srcs_a, dsts_a = [], [], [], []
    for ei in (edge_index_seq, edge_index_knn, edge_index_dis):
        sflat = jnp.concatenate([ei[0], pad])
        dflat = jnp.concatenate([ei[1], pad])
        srcs.append(sflat.reshape(NC, NS, NCHUNK, CHUNK))
        dsts.append(dflat.reshape(NC, NS, NCHUNK, CHUNK))
        srcs_a.append(_asym(sflat))
        dsts_a.append(_asym(dflat))
    zeros_h = jnp.zeros((CHUNK, D), jnp.float32)

    scl = _scales(_counts6(srcs, dsts, zeros_h))

    y0, y1, y2 = _premm(xp, scl, W0_seq, W0_knn, W0_dis)
    a0, a1, a2 = _scatter3(y0, y1, y2, srcs_a, dsts_a, zeros_h)
    u, stats = _fc_stage(a0, a1, a2, scl, b0_seq, b0_knn, b0_dis, Wfc0, bfc0)

    y0, y1, y2 = _bn_premm(u, stats, gamma0, beta0, scl,
                           W1_seq, W1_knn, W1_dis)
    a0, a1, a2 = _scatter3(y0, y1, y2, srcs_a, dsts_a, zeros_h)
    u, stats = _fc_stage(a0, a1, a2, scl, b1_seq, b1_knn, b1_dis, Wfc1, bfc1)

    return _bn_final(u, stats, gamma1, beta1)


# speedup vs baseline: 1.1432x; 1.1432x over previous
import jax
import jax.numpy as jnp
from jax import lax
from jax.experimental import pallas as pl
from jax.experimental.pallas import tpu as pltpu
from jax.experimental.pallas import tpu_sc as plsc

N = 10000
D = 128
E = 160000
NC = 2
NS = 16
CHUNK = 128
NCHUNK = 40
EPAD = NC * NS * NCHUNK * CHUNK
NPAD = 10240
RPW = NPAD // NS
BT = 1024
PK = 2
CH0 = 60
CH1 = 20
EPS = 1e-5


def _sc_mesh():
    return plsc.VectorSubcoreMesh(core_axis_name="c", subcore_axis_name="s")


def _counts6(srcs, dsts, zeros_h):
    def body(i0, i1, i2, i3, i4, i5, eye_h, zeros_h, cnt_out,
             idx_a, ones_v, rows, acc, ssem):
        c = lax.axis_index("c")
        s = lax.axis_index("s")
        idxs = (i0, i1, i2, i3, i4, i5)
        pltpu.sync_copy(zeros_h, rows)
        for t in range(RPW // CHUNK):
            pltpu.sync_copy(rows, acc.at[pl.ds(s * RPW + t * CHUNK, CHUNK)])
        plsc.subcore_barrier()
        for k in range(6):
            pltpu.sync_copy(eye_h.at[k], ones_v)
            pltpu.sync_copy(idxs[k].at[c, s], idx_a)

            def grp_body(g, carry, k=k):
                sds = [pltpu.async_copy(ones_v, acc.at[idx_a.at[g * PK + q]],
                                        ssem, add=True)
                       for q in range(PK)]
                for sd in sds:
                    sd.wait()
                return carry

            lax.fori_loop(0, NCHUNK // PK, grp_body, 0)
        plsc.subcore_barrier()
        for t in range(RPW // CHUNK):
            rowsl = pl.ds(s * RPW + t * CHUNK, CHUNK)
            pltpu.sync_copy(acc.at[rowsl], rows)
            pltpu.sync_copy(rows, cnt_out.at[c, rowsl])

    f = pl.kernel(
        body,
        out_type=jax.ShapeDtypeStruct((NC, NPAD, D), jnp.float32),
        mesh=_sc_mesh(),
        scratch_types=[
            pltpu.VMEM((NCHUNK, CHUNK), jnp.int32),
            pltpu.VMEM((CHUNK, D), jnp.float32),
            pltpu.VMEM((CHUNK, D), jnp.float32),
            pltpu.VMEM_SHARED((NPAD, D), jnp.float32),
            pltpu.SemaphoreType.DMA,
        ],
    )
    mask = jnp.repeat(jnp.eye(6, 8, dtype=jnp.float32), 16, axis=1)
    eye_h = jnp.broadcast_to(mask[:, None, :], (6, CHUNK, D))
    return f(srcs[0], dsts[0], srcs[1], dsts[1], srcs[2], dsts[2],
             eye_h, zeros_h)


def _scatter3(y0, y1, y2, srcs, dsts, zeros_h):
    def body(y0, y1, y2, s0, d0, s1, d1, s2, d2, zeros_h, o0, o1, o2,
             idx_sa, idx_da, r0, r1, acc,
             g0, g1, ssem):
        c = lax.axis_index("c")
        s = lax.axis_index("s")
        ys = (y0, y1, y2)
        sps = (s0, s1, s2)
        dps = (d0, d1, d2)
        outs = (o0, o1, o2)
        rbufs = (r0, r1)
        gsems = (g0, g1)
        for r in range(3):
            pltpu.sync_copy(zeros_h, r0)
            for t in range(RPW // CHUNK):
                pltpu.sync_copy(r0,
                                acc.at[pl.ds(s * RPW + t * CHUNK, CHUNK)])
            plsc.subcore_barrier()
            pltpu.sync_copy(sps[r].at[c, s], idx_sa)
            pltpu.sync_copy(dps[r].at[c, s], idx_da)

            def grp_body(g, carry, r=r):
                gds = [pltpu.async_copy(ys[r].at[idx_sa.at[g * PK + q]],
                                        rbufs[q], gsems[q])
                       for q in range(PK)]
                sds = []
                for q in range(PK):
                    gds[q].wait()
                    sds.append(pltpu.async_copy(
                        rbufs[q], acc.at[idx_da.at[g * PK + q]],
                        ssem, add=True))
                for sd in sds:
                    sd.wait()
                return carry

            ngrp = jnp.where(c == 0, CH0 // PK, CH1 // PK)
            lax.fori_loop(0, ngrp, grp_body, 0)
            plsc.subcore_barrier()
            for t in range(RPW // CHUNK):
                rowsl = pl.ds(s * RPW + t * CHUNK, CHUNK)
                pltpu.sync_copy(acc.at[rowsl], r0)
                pltpu.sync_copy(r0, outs[r].at[c, rowsl])

    f = pl.kernel(
        body,
        out_type=[jax.ShapeDtypeStruct((NC, NPAD, D), jnp.float32)] * 3,
        mesh=_sc_mesh(),
        scratch_types=[
            pltpu.VMEM((CH0, CHUNK), jnp.int32),
            pltpu.VMEM((CH0, CHUNK), jnp.int32),
            pltpu.VMEM((CHUNK, D), jnp.float32),
            pltpu.VMEM((CHUNK, D), jnp.float32),
            pltpu.VMEM_SHARED((NPAD, D), jnp.float32),
            pltpu.SemaphoreType.DMA,
            pltpu.SemaphoreType.DMA,
            pltpu.SemaphoreType.DMA,
        ],
    )
    return f(y0, y1, y2, srcs[0], dsts[0], srcs[1], dsts[1],
             srcs[2], dsts[2], zeros_h)


_BLK = pl.BlockSpec((BT, D), lambda i: (i, 0))
_SCL = pl.BlockSpec((6, BT, 16), lambda i: (0, i, 0))
_FULL = pl.BlockSpec(memory_space=pltpu.VMEM)


def _scales_body(cnt_ref, scl_ref):
    tot = cnt_ref[0] + cnt_ref[1]
    for k in range(6):
        scl_ref[k] = lax.rsqrt(jnp.maximum(tot[:, 16 * k:16 * (k + 1)], 1.0))


def _scales(cnt):
    return pl.pallas_call(
        _scales_body,
        grid=(NPAD // BT,),
        in_specs=[pl.BlockSpec((NC, BT, D), lambda i: (0, i, 0))],
        out_specs=_SCL,
        out_shape=jax.ShapeDtypeStruct((6, NPAD, 16), jnp.float32),
    )(cnt)


def _premm_body(h_ref, scl_ref, w0_ref, w1_ref, w2_ref,
                y0_ref, y1_ref, y2_ref):
    h = h_ref[...]
    for r, (w_ref, y_ref) in enumerate(((w0_ref, y0_ref),
                                        (w1_ref, y1_ref),
                                        (w2_ref, y2_ref))):
        y_ref[...] = jnp.dot(h * scl_ref[2 * r, :, 0:1], w_ref[...],
                             preferred_element_type=jnp.float32,
                             precision=lax.Precision.HIGHEST)


def _premm(h, scl, w0, w1, w2):
    shp = jax.ShapeDtypeStruct((NPAD, D), jnp.float32)
    return pl.pallas_call(
        _premm_body,
        grid=(NPAD // BT,),
        in_specs=[_BLK, _SCL, _FULL, _FULL, _FULL],
        out_specs=[_BLK, _BLK, _BLK],
        out_shape=[shp, shp, shp],
    )(h, scl, w0, w1, w2)


def _fc_body(a0_ref, a1_ref, a2_ref, scl_ref, b0_ref, b1_ref, b2_ref,
             wfc_ref, bfc_ref, u_ref, stats_ref, sacc_ref):
    i = pl.program_id(0)
    z = (b0_ref[...] + b1_ref[...] + b2_ref[...])[None, :]
    for r, a_ref in enumerate((a0_ref, a1_ref, a2_ref)):
        z = z + (a_ref[0] + a_ref[1]) * scl_ref[2 * r + 1, :, 0:1]
    u = jnp.dot(z, wfc_ref[...], preferred_element_type=jnp.float32,
                             precision=lax.Precision.HIGHEST)
    u = jnp.maximum(u + bfc_ref[...][None, :], 0.0)
    u_ref[...] = u
    ids = i * BT + lax.broadcasted_iota(jnp.int32, (BT, 1), 0)
    um = jnp.where(ids < N, u, 0.0)

    @pl.when(i == 0)
    def _():
        sacc_ref[...] = jnp.zeros_like(sacc_ref)

    sacc_ref[0:1, :] += jnp.sum(um, axis=0, keepdims=True)
    sacc_ref[1:2, :] += jnp.sum(um * um, axis=0, keepdims=True)

    @pl.when(i == pl.num_programs(0) - 1)
    def _():
        stats_ref[...] = sacc_ref[...]


def _fc_stage(a0, a1, a2, scl, b0, b1, b2, wfc, bfc):
    acc_spec = pl.BlockSpec((NC, BT, D), lambda i: (0, i, 0))
    return pl.pallas_call(
        _fc_body,
        grid=(NPAD // BT,),
        in_specs=[acc_spec, acc_spec, acc_spec, _SCL,
                  _FULL, _FULL, _FULL, _FULL, _FULL],
        out_specs=[_BLK, pl.BlockSpec((8, 128), lambda i: (0, 0))],
        out_shape=[jax.ShapeDtypeStruct((NPAD, D), jnp.float32),
                   jax.ShapeDtypeStruct((8, 128), jnp.float32)],
        scratch_shapes=[pltpu.VMEM((8, 128), jnp.float32)],
    )(a0, a1, a2, scl, b0, b1, b2, wfc, bfc)


def _bn(u, stats, gamma, beta):
    mean = stats[0:1, :] * (1.0 / N)
    var = stats[1:2, :] * (1.0 / N) - mean * mean
    return gamma[None, :] * (u - mean) * lax.rsqrt(var + EPS) + beta[None, :]


def _bn_premm_body(u_ref, stats_ref, g_ref, be_ref, scl_ref,
                   w0_ref, w1_ref, w2_ref, y0_ref, y1_ref, y2_ref):
    h = _bn(u_ref[...], stats_ref[...], g_ref[...], be_ref[...])
    for r, (w_ref, y_ref) in enumerate(((w0_ref, y0_ref),
                                        (w1_ref, y1_ref),
                                        (w2_ref, y2_ref))):
        y_ref[...] = jnp.dot(h * scl_ref[2 * r, :, 0:1], w_ref[...],
                             preferred_element_type=jnp.float32,
                             precision=lax.Precision.HIGHEST)


def _bn_premm(u, stats, gamma, beta, scl, w0, w1, w2):
    shp = jax.ShapeDtypeStruct((NPAD, D), jnp.float32)
    return pl.pallas_call(
        _bn_premm_body,
        grid=(NPAD // BT,),
        in_specs=[_BLK, _FULL, _FULL, _FULL, _SCL, _FULL, _FULL, _FULL],
        out_specs=[_BLK, _BLK, _BLK],
        out_shape=[shp, shp, shp],
    )(u, stats, gamma, beta, scl, w0, w1, w2)


def _bn_final_body(u_ref, stats_ref, g_ref, be_ref, out_ref):
    out_ref[...] = _bn(u_ref[...], stats_ref[...], g_ref[...], be_ref[...])


def _bn_final(u, stats, gamma, beta):
    blk = pl.BlockSpec((1000, D), lambda i: (i, 0))
    return pl.pallas_call(
        _bn_final_body,
        grid=(N // 1000,),
        in_specs=[blk, _FULL, _FULL, _FULL],
        out_specs=blk,
        out_shape=jax.ShapeDtypeStruct((N, D), jnp.float32),
    )(u, stats, gamma, beta)


def kernel(x, edge_index_seq, edge_index_knn, edge_index_dis,
           W0_seq, b0_seq, W0_knn, b0_knn, W0_dis, b0_dis, Wfc0, bfc0,
           gamma0, beta0,
           W1_seq, b1_seq, W1_knn, b1_knn, W1_dis, b1_dis, Wfc1, bfc1,
           gamma1, beta1):
    xp = jnp.pad(x, ((0, NPAD - N), (0, 0)))
    pad = jnp.full((EPAD - E,), N, jnp.int32)
    e0 = NS * CH0 * CHUNK

    def _asym(flat):
        a0 = flat[:e0].reshape(NS, CH0, CHUNK)
        a1 = flat[e0:].reshape(NS, CH1, CHUNK)
        a1 = jnp.concatenate(
            [a1, jnp.zeros((NS, CH0 - CH1, CHUNK), jnp.int32)], axis=1)
        return jnp.stack([a0, a1])

    srcs, dsts, srcs_a, dsts_a = [], [], [], []
    for ei in (edge_index_seq, edge_index_knn, edge_index_dis):
        sflat = jnp.concatenate([ei[0], pad])
        dflat = jnp.concatenate([ei[1], pad])
        srcs.append(sflat.reshape(NC, NS, NCHUNK, CHUNK))
        dsts.append(dflat.reshape(NC, NS, NCHUNK, CHUNK))
        srcs_a.append(_asym(sflat))
        dsts_a.append(_asym(dflat))
    zeros_h = jnp.zeros((CHUNK, D), jnp.float32)

    scl = _scales(_counts6(srcs, dsts, zeros_h))

    y0, y1, y2 = _premm(xp, scl, W0_seq, W0_knn, W0_dis)
    a0, a1, a2 = _scatter3(y0, y1, y2, srcs_a, dsts_a, zeros_h)
    u, stats = _fc_stage(a0, a1, a2, scl, b0_seq, b0_knn, b0_dis, Wfc0, bfc0)

    y0, y1, y2 = _bn_premm(u, stats, gamma0, beta0, scl,
                           W1_seq, W1_knn, W1_dis)
    a0, a1, a2 = _scatter3(y0, y1, y2, srcs_a, dsts_a, zeros_h)
    u, stats = _fc_stage(a0, a1, a2, scl, b1_seq, b1_knn, b1_dis, Wfc1, bfc1)

    return _bn_final(u, stats, gamma1, beta1)
